# baseline (speedup 1.0000x reference)
import functools

import jax
import jax.numpy as jnp
from jax import lax
from jax.experimental import pallas as pl
from jax.experimental.pallas import tpu as pltpu
from jax.experimental.pallas import tpu_sc as plsc

_NBINS = 32
_FEATS = 128
_LANES = 16
_NWORKERS = 32
_ROWS_PER_BLOCK = 256
_VECS_PER_ROW = _FEATS // _LANES


def _tokenizer_body(x_hbm, bnd_hbm, out_hbm, xbuf, hist, bnd):
    rows_total = x_hbm.shape[0] // _FEATS
    rows_per_worker = rows_total // _NWORKERS
    n_blocks = rows_per_worker // _ROWS_PER_BLOCK

    wid = lax.axis_index("c") * 16 + lax.axis_index("s")
    row0 = wid * rows_per_worker

    pltpu.sync_copy(bnd_hbm, bnd)

    val16 = jnp.full((_LANES,), 2.0 ** -7, jnp.float32)
    zeros16 = jnp.zeros((_LANES,), jnp.float32)

    def zero_body(i, _):
        hist[pl.ds(i * _LANES, _LANES)] = zeros16
        return 0

    def row_body(r, _):
        rbase = r * _FEATS
        obase = r * _NBINS
        for j in range(_VECS_PER_ROW):
            x16 = xbuf[pl.ds(rbase + j * _LANES, _LANES)]
            t = x16 * jnp.float32(5.0) + jnp.float32(15.5)
            t = jnp.minimum(jnp.maximum(t, jnp.float32(0.0)),
                            jnp.float32(30.0))
            g = t.astype(jnp.int32)
            bg = plsc.load_gather(bnd, [g])
            y = g + (x16 >= bg).astype(jnp.int32)
            plsc.addupdate_scatter(hist, [y + obase], val16)
        return 0

    for blk in range(n_blocks):
        blk_row = row0 + blk * _ROWS_PER_BLOCK
        pltpu.sync_copy(
            x_hbm.at[pl.ds(blk_row * _FEATS, _ROWS_PER_BLOCK * _FEATS)],
            xbuf)
        lax.fori_loop(0, _ROWS_PER_BLOCK * _NBINS // _LANES, zero_body, 0)
        lax.fori_loop(0, _ROWS_PER_BLOCK, row_body, 0)
        pltpu.sync_copy(
            hist,
            out_hbm.at[pl.ds(blk_row * _NBINS,
                             _ROWS_PER_BLOCK * _NBINS)])


def kernel(x):
    B, T, F = x.shape
    rows = B * T
    x_flat = x.reshape(rows * F)
    bnd = jnp.linspace(-3.0, 3.0, _NBINS - 1).astype(jnp.float32)
    bnd = jnp.concatenate([bnd, jnp.full((1,), 3.0, jnp.float32)])

    mesh = plsc.VectorSubcoreMesh(core_axis_name="c", subcore_axis_name="s")
    run = functools.partial(
        pl.kernel,
        out_type=jax.ShapeDtypeStruct((rows * _NBINS,), jnp.float32),
        mesh=mesh,
        compiler_params=pltpu.CompilerParams(needs_layout_passes=False),
        scratch_types=[
            pltpu.VMEM((_ROWS_PER_BLOCK * _FEATS,), jnp.float32),
            pltpu.VMEM((_ROWS_PER_BLOCK * _NBINS,), jnp.float32),
            pltpu.VMEM((_NBINS,), jnp.float32),
        ],
    )(_tokenizer_body)
    out = run(x_flat, bnd)
    return out.reshape(B, T, _NBINS)


# speedup vs baseline: 3.4580x; 3.4580x over previous
import functools

import jax
import jax.numpy as jnp
from jax import lax
from jax.experimental import pallas as pl
from jax.experimental.pallas import tpu as pltpu
from jax.experimental.pallas import tpu_sc as plsc

_NBINS = 32
_FEATS = 128
_LANES = 16
_NWORKERS = 32
_ROWS_PER_BLOCK = 256
_VECS_PER_ROW = _FEATS // _LANES


def _tokenizer_body(x_hbm, bnd_hbm, out_hbm, xbuf, hist, bnd):
    rows_total = x_hbm.shape[0] // _FEATS
    rows_per_worker = rows_total // _NWORKERS
    n_blocks = rows_per_worker // _ROWS_PER_BLOCK

    wid = lax.axis_index("c") * 16 + lax.axis_index("s")
    row0 = wid * rows_per_worker

    pltpu.sync_copy(bnd_hbm, bnd)

    val16 = jnp.full((_LANES,), 2.0 ** -7, jnp.float32)
    zeros16 = jnp.zeros((_LANES,), jnp.float32)

    def zero_body(i):
        hist[pl.ds(i * _LANES, _LANES)] = zeros16

    def row_body(r):
        rbase = r * _FEATS
        obase = r * _NBINS
        for j in range(_VECS_PER_ROW):
            x16 = xbuf[pl.ds(rbase + j * _LANES, _LANES)]
            t = x16 * jnp.float32(5.0) + jnp.float32(15.5)
            t = jnp.minimum(jnp.maximum(t, jnp.float32(0.0)),
                            jnp.float32(30.0))
            g = t.astype(jnp.int32)
            bg = plsc.load_gather(bnd, [g])
            y = g + (x16 >= bg).astype(jnp.int32)
            plsc.addupdate_scatter(hist, [y + obase], val16)

    def blk_body(blk, _):
        blk_row = row0 + blk * _ROWS_PER_BLOCK
        pltpu.sync_copy(
            x_hbm.at[pl.ds(blk_row * _FEATS, _ROWS_PER_BLOCK * _FEATS)],
            xbuf)
        plsc.parallel_loop(0, _ROWS_PER_BLOCK * _NBINS // _LANES,
                           unroll=4)(zero_body)
        plsc.parallel_loop(0, _ROWS_PER_BLOCK, unroll=4)(row_body)
        pltpu.sync_copy(
            hist,
            out_hbm.at[pl.ds(blk_row * _NBINS,
                             _ROWS_PER_BLOCK * _NBINS)])
        return 0

    lax.fori_loop(0, n_blocks, blk_body, 0)


def kernel(x):
    B, T, F = x.shape
    rows = B * T
    x_flat = x.reshape(rows * F)
    bnd = jnp.linspace(-3.0, 3.0, _NBINS - 1).astype(jnp.float32)
    bnd = jnp.concatenate([bnd, jnp.full((1,), 3.0, jnp.float32)])

    mesh = plsc.VectorSubcoreMesh(core_axis_name="c", subcore_axis_name="s")
    run = functools.partial(
        pl.kernel,
        out_type=jax.ShapeDtypeStruct((rows * _NBINS,), jnp.float32),
        mesh=mesh,
        compiler_params=pltpu.CompilerParams(needs_layout_passes=False),
        scratch_types=[
            pltpu.VMEM((_ROWS_PER_BLOCK * _FEATS,), jnp.float32),
            pltpu.VMEM((_ROWS_PER_BLOCK * _NBINS,), jnp.float32),
            pltpu.VMEM((_NBINS,), jnp.float32),
        ],
    )(_tokenizer_body)
    out = run(x_flat, bnd)
    return out.reshape(B, T, _NBINS)
